# baseline (speedup 1.0000x reference)
import functools

import jax
import jax.numpy as jnp
from jax import lax
from jax.experimental import pallas as pl
from jax.experimental.pallas import tpu as pltpu
from jax.experimental.pallas import tpu_sc as plsc

_B = 4096
_GT, _GN, _GD = 13, 100000, 64
_CT, _CN, _CD = 13, 1000000, 32
_LANES = 128

_NC, _NS = 2, 16
_NW = _NC * _NS
_BPW = _B // _NW
_RPW = _GT * _BPW


def _sc_gather(gt_wide, gidx, ct_wide, cidx):
    mesh = plsc.VectorSubcoreMesh(
        core_axis_name="c", subcore_axis_name="s",
        num_cores=_NC, num_subcores=_NS)

    @functools.partial(
        pl.kernel,
        out_type=(jax.ShapeDtypeStruct((_GT, _B, _LANES), jnp.float32),
                  jax.ShapeDtypeStruct((_CT, _B, _LANES), jnp.float32)),
        mesh=mesh,
        scratch_types=(
            pltpu.VMEM((1, _RPW), jnp.int32),
            pltpu.VMEM((1, _RPW), jnp.int32),
            pltpu.VMEM((_BPW, _LANES), jnp.float32),
            pltpu.VMEM((_BPW, _LANES), jnp.float32),
            pltpu.SemaphoreType.DMA,
            pltpu.SemaphoreType.DMA,
        ),
    )
    def k(gt_hbm, gidx_hbm, ct_hbm, cidx_hbm, gout_hbm, cout_hbm,
          gi_v, ci_v, buf0, buf1, sem0, sem1):
        wid = lax.axis_index("s") * _NC + lax.axis_index("c")
        rbase = wid * _BPW
        pltpu.sync_copy(gidx_hbm.at[wid], gi_v)
        pltpu.sync_copy(cidx_hbm.at[wid], ci_v)
        bufs = (buf0, buf1)
        sems = (sem0, sem1)
        jobs = ([(gt_hbm, gi_v, gout_hbm, t) for t in range(_GT)]
                + [(ct_hbm, ci_v, cout_hbm, t) for t in range(_CT)])
        copies = [None, None]
        for n, (tab, idx_v, out, t) in enumerate(jobs):
            p = n & 1
            if copies[p] is not None:
                copies[p].wait()
            idx = idx_v.at[0, pl.ds(t * _BPW, _BPW)]
            pltpu.async_copy(tab.at[idx], bufs[p], sems[p]).wait()
            copies[p] = pltpu.async_copy(
                bufs[p], out.at[t, pl.ds(rbase, _BPW)], sems[p])
        copies[0].wait()
        copies[1].wait()

    return k(gt_wide, gidx, ct_wide, cidx)


def _mlp_body(df, gp, cp, hg, hc, dw1t, db1, dw2t, db2,
              w1gd, w1cd, w1dt, ob1, ow2t, ob2, ow3t, ob3, ow4t, ob4,
              ow5t, ob5, out):
    dot = functools.partial(jnp.dot, preferred_element_type=jnp.float32)
    bb = df.shape[0]
    h = jnp.maximum(dot(df[...], dw1t[...]) + db1[...], 0.0)
    de = dot(h, dw2t[...]) + db2[...]
    o = dot(de, w1dt[...]) + ob1[...]
    lane = lax.broadcasted_iota(jnp.int32, (bb, _LANES), 1)
    hgv = hg[...]
    hcv = hc[...]
    for t in range(_GT):
        m = (lane >> 6) == hgv[t][:, None]
        o = o + dot(jnp.where(m, gp[t], 0.0), w1gd[t])
    for t in range(_CT):
        m = (lane >> 5) == hcv[t][:, None]
        o = o + dot(jnp.where(m, cp[t], 0.0), w1cd[t])
    o = jnp.maximum(o, 0.0)
    o = jnp.maximum(dot(o, ow2t[...]) + ob2[...], 0.0)
    o = jnp.maximum(dot(o, ow3t[...]) + ob3[...], 0.0)
    o = jnp.maximum(dot(o, ow4t[...]) + ob4[...], 0.0)
    out[...] = dot(o, ow5t[...]) + ob5[...]


def _tc_mlp(df, gp3, cp3, hgt, hct, dw1t, db1, dw2t, db2,
            w1gd, w1cd, w1dt, ob1, ow2t, ob2, ow3t, ob3, ow4t, ob4,
            ow5t, ob5, block_b=512):
    grid = (_B // block_b,)

    def full_spec(a):
        return pl.BlockSpec(a.shape, lambda i: (0,) * a.ndim)

    weights = (dw1t, db1, dw2t, db2, w1gd, w1cd, w1dt, ob1,
               ow2t, ob2, ow3t, ob3, ow4t, ob4, ow5t, ob5)
    return pl.pallas_call(
        _mlp_body,
        grid=grid,
        in_specs=[pl.BlockSpec((block_b, df.shape[1]), lambda i: (i, 0)),
                  pl.BlockSpec((_GT, block_b, _LANES), lambda i: (0, i, 0)),
                  pl.BlockSpec((_CT, block_b, _LANES), lambda i: (0, i, 0)),
                  pl.BlockSpec((_GT, block_b), lambda i: (0, i)),
                  pl.BlockSpec((_CT, block_b), lambda i: (0, i))]
                 + [full_spec(w) for w in weights],
        out_specs=pl.BlockSpec((block_b, 1), lambda i: (i, 0)),
        out_shape=jax.ShapeDtypeStruct((_B, 1), jnp.float32),
    )(df, gp3, cp3, hgt, hct, *weights)


def _worker_major(idx):
    return (idx.T.reshape(idx.shape[1], _NW, _BPW)
            .transpose(1, 0, 2).reshape(_NW, 1, -1))


def kernel(dense_features, gpu_sharded_sparse_features, cpu_sharded_sparse_features,
           gpu_tables, cpu_tables, dw1, db1, dw2, db2,
           ow1, ob1, ow2, ob2, ow3, ob3, ow4, ob4, ow5, ob5):
    gflat = (gpu_sharded_sparse_features.astype(jnp.int32)
             + jnp.arange(_GT, dtype=jnp.int32)[None, :] * _GN)
    cflat = (cpu_sharded_sparse_features.astype(jnp.int32)
             + jnp.arange(_CT, dtype=jnp.int32)[None, :] * _CN)
    gidx = _worker_major(gflat >> 1)
    cidx = _worker_major(cflat >> 2)
    hgt = (gflat & 1).T
    hct = (cflat & 3).T

    gp3, cp3 = _sc_gather(
        gpu_tables.reshape(_GT * _GN // 2, _LANES), gidx,
        cpu_tables.reshape(_CT * _CN // 4, _LANES), cidx)

    ow1t = ow1.T
    g_cols = _GT * _GD
    c_cols = _CT * _CD
    w1gd = jnp.broadcast_to(
        ow1t[:g_cols].reshape(_GT, 1, _GD, 512),
        (_GT, _LANES // _GD, _GD, 512)).reshape(_GT, _LANES, 512)
    w1cd = jnp.broadcast_to(
        ow1t[g_cols:g_cols + c_cols].reshape(_CT, 1, _CD, 512),
        (_CT, _LANES // _CD, _CD, 512)).reshape(_CT, _LANES, 512)
    w1dt = ow1t[g_cols + c_cols:]

    return _tc_mlp(
        dense_features, gp3, cp3, hgt, hct,
        dw1.T, db1[None, :], dw2.T, db2[None, :],
        w1gd, w1cd, w1dt, ob1[None, :],
        ow2.T, ob2[None, :], ow3.T, ob3[None, :], ow4.T, ob4[None, :],
        ow5.T, ob5[None, :])
